# baseline (speedup 1.0000x reference)
import functools

import jax
import jax.numpy as jnp
import numpy as np
from jax import lax
from jax.experimental import pallas as pl
from jax.experimental.pallas import tpu as pltpu
from jax.experimental.pallas import tpu_sc as plsc

E = 64
D = 1024
F = 2048
CAP_FACTOR = 1.25
NC, NS = 2, 16
NW = NC * NS


def _make_routing_dispatch(T, C, C2, e_base, chunk):
  per_w = C2
  n_ch = per_w // chunk
  nv = T // 16
  mesh = plsc.VectorSubcoreMesh(core_axis_name="c", subcore_axis_name="s")

  @functools.partial(
      pl.kernel, mesh=mesh,
      compiler_params=pltpu.CompilerParams(needs_layout_passes=False),
      out_type=(jax.ShapeDtypeStruct((NW * C2, D), jnp.float32),
                jax.ShapeDtypeStruct((NW, T), jnp.int32)),
      scratch_types=[
          pltpu.VMEM((T,), jnp.int32),
          pltpu.VMEM((T,), jnp.int32),
          pltpu.VMEM((per_w,), jnp.int32),
          pltpu.VMEM((chunk, D), jnp.float32),
          pltpu.VMEM((chunk, D), jnp.float32),
          pltpu.SemaphoreType.DMA,
          pltpu.SemaphoreType.DMA,
      ],
  )
  def rd(ids_hbm, x_hbm, buf_hbm, destp_hbm, ids_v, destc_v, idx_v,
         rows0, rows1, sem0, sem1):
    cid = lax.axis_index("c")
    sid = lax.axis_index("s")
    w = sid * NC + cid
    base = w * per_w
    e0 = e_base + w

    pltpu.sync_copy(ids_hbm, ids_v)

    lanes = lax.iota(jnp.int32, 16)

    def bcast(s):
      return jnp.full((16,), s, jnp.int32)

    def init_body(i, _):
      o = pl.multiple_of(i * 16, 16)
      iv = lanes + bcast(base + i * 16)
      idx_v[pl.ds(o, 16)] = lax.rem(iv, bcast(T))
      return 0

    lax.fori_loop(0, per_w // 16, init_body, 0)

    ec0 = bcast(E)
    e0v = bcast(e0)
    onev = bcast(1)
    cv = bcast(C)
    d0base = bcast(e0 * C2 + 1)
    zero = bcast(0)

    dn = lax.GatherDimensionNumbers(offset_dims=(),
                                    collapsed_slice_dims=(0,),
                                    start_index_map=(0,))

    def lgather(s, idx):
      return lax.gather(s, idx[:, None], dn, slice_sizes=(1,),
                        mode=lax.GatherScatterMode.PROMISE_IN_BOUNDS)

    lane15 = bcast(15)

    def body(i, c0v):
      o = pl.multiple_of(i * 16, 16)
      v = ids_v[pl.ds(o, 16)]
      ex = lax.rem(v, ec0)
      tok = lanes + bcast(i * 16)
      m0 = ex == e0v
      im0 = jnp.where(m0, onev, zero)
      p0 = plsc.cumsum(im0)
      s0 = c0v + p0 - onev
      k0 = jnp.logical_and(m0, s0 < cv)
      plsc.store_scatter(idx_v, [s0], tok, mask=k0)
      d0 = d0base + jnp.minimum(s0, cv)
      destc_v[pl.ds(o, 16)] = jnp.where(m0, d0, zero)
      return c0v + lgather(p0, lane15)

    lax.fori_loop(0, nv, body, zero)

    pltpu.sync_copy(destc_v, destp_hbm.at[w])

    rows = (rows0, rows1)
    sems = (sem0, sem1)
    cps = [None, None]
    cps[0] = pltpu.async_copy(x_hbm.at[idx_v.at[pl.ds(0, chunk)]], rows0, sem0)
    for k in range(n_ch):
      if k + 1 < n_ch:
        cps[(k + 1) % 2] = pltpu.async_copy(
            x_hbm.at[idx_v.at[pl.ds((k + 1) * chunk, chunk)]],
            rows[(k + 1) % 2], sems[(k + 1) % 2])
      cps[k % 2].wait()
      pltpu.sync_copy(rows[k % 2], buf_hbm.at[pl.ds(base + k * chunk, chunk)])

  return rd


def _ffn_body(C, C2, buf_ref, w1_ref, w2_ref, y_ref):
  f = pl.program_id(1)
  b = buf_ref[0]
  h = jnp.maximum(
      lax.dot_general(b, w1_ref[0], (((1,), (0,)), ((), ())),
                      preferred_element_type=jnp.float32), 0.0)
  yp = lax.dot_general(h, w2_ref[0], (((1,), (0,)), ((), ())),
                       preferred_element_type=jnp.float32)
  rowmask = (lax.broadcasted_iota(jnp.int32, (C2, 1), 0) < C).astype(yp.dtype)
  yp = yp * rowmask

  @pl.when(f == 0)
  def _():
    y_ref[0] = yp

  @pl.when(f != 0)
  def _():
    y_ref[0] = y_ref[0] + yp


def _ffn_body_alias(C, C2, yin_ref, buf_ref, w1_ref, w2_ref, y_ref):
  del yin_ref
  _ffn_body(C, C2, buf_ref, w1_ref, w2_ref, y_ref)


def _make_ffn(C, C2, fblk, e_base):
  nf = F // fblk
  ne = E // 2
  return pl.pallas_call(
      functools.partial(_ffn_body, C, C2),
      grid=(ne, nf),
      in_specs=[
          pl.BlockSpec((1, C2, D), lambda e, f: (e, 0, 0)),
          pl.BlockSpec((1, D, fblk), lambda e, f: (e + e_base, 0, f)),
          pl.BlockSpec((1, fblk, D), lambda e, f: (e + e_base, f, 0)),
      ],
      out_specs=pl.BlockSpec((1, C2, D), lambda e, f: (e + e_base, 0, 0)),
      out_shape=jax.ShapeDtypeStruct((E, C2, D), jnp.float32),
      compiler_params=pltpu.CompilerParams(
          dimension_semantics=("arbitrary", "arbitrary")),
  )


def _make_ffn_alias(C, C2, fblk, e_base):
  nf = F // fblk
  ne = E // 2
  return pl.pallas_call(
      functools.partial(_ffn_body_alias, C, C2),
      grid=(ne, nf),
      in_specs=[
          pl.BlockSpec(memory_space=pl.ANY),
          pl.BlockSpec((1, C2, D), lambda e, f: (e, 0, 0)),
          pl.BlockSpec((1, D, fblk), lambda e, f: (e + e_base, 0, f)),
          pl.BlockSpec((1, fblk, D), lambda e, f: (e + e_base, f, 0)),
      ],
      out_specs=pl.BlockSpec((1, C2, D), lambda e, f: (e + e_base, 0, 0)),
      out_shape=jax.ShapeDtypeStruct((E, C2, D), jnp.float32),
      input_output_aliases={0: 0},
      compiler_params=pltpu.CompilerParams(
          dimension_semantics=("arbitrary", "arbitrary")),
  )


def _make_combine(T, chunk):
  per_w = T // NW
  n_ch = per_w // chunk
  mesh = plsc.VectorSubcoreMesh(core_axis_name="c", subcore_axis_name="s")

  @functools.partial(
      pl.kernel, mesh=mesh,
      compiler_params=pltpu.CompilerParams(needs_layout_passes=False),
      out_type=jax.ShapeDtypeStruct((T, D), jnp.float32),
      scratch_types=[
          pltpu.VMEM((NW, per_w), jnp.int32),
          pltpu.VMEM((NW, per_w), jnp.int32),
          pltpu.VMEM((per_w,), jnp.int32),
          pltpu.VMEM((chunk, D), jnp.float32),
          pltpu.VMEM((chunk, D), jnp.float32),
          pltpu.SemaphoreType.DMA,
          pltpu.SemaphoreType.DMA,
      ],
  )
  def combine(y_hbm, destpa_hbm, destpb_hbm, out_hbm, dpa_v, dpb_v, idx_v,
              rows0, rows1, sem0, sem1):
    wid = lax.axis_index("s") * NC + lax.axis_index("c")
    base = wid * per_w
    pltpu.sync_copy(destpa_hbm.at[:, pl.ds(base, per_w)], dpa_v)
    pltpu.sync_copy(destpb_hbm.at[:, pl.ds(base, per_w)], dpb_v)

    def sum_body(i, _):
      o = pl.multiple_of(i * 16, 16)
      acc = jnp.full((16,), -1, jnp.int32)
      for s2 in range(NW):
        acc = acc + dpa_v[s2, pl.ds(o, 16)]
        acc = acc + dpb_v[s2, pl.ds(o, 16)]
      idx_v[pl.ds(o, 16)] = acc
      return 0

    lax.fori_loop(0, per_w // 16, sum_body, 0)

    rows = (rows0, rows1)
    sems = (sem0, sem1)
    cps = [None, None]
    cps[0] = pltpu.async_copy(y_hbm.at[idx_v.at[pl.ds(0, chunk)]], rows0, sem0)
    for k in range(n_ch):
      if k + 1 < n_ch:
        cps[(k + 1) % 2] = pltpu.async_copy(
            y_hbm.at[idx_v.at[pl.ds((k + 1) * chunk, chunk)]],
            rows[(k + 1) % 2], sems[(k + 1) % 2])
      cps[k % 2].wait()
      pltpu.sync_copy(rows[k % 2], out_hbm.at[pl.ds(base + k * chunk, chunk)])

  return combine


@jax.jit
def kernel(hidden_states, input_ids, w1, w2):
  B, S, _ = hidden_states.shape
  T = B * S
  C = int(np.ceil(T / E * CAP_FACTOR))
  C2 = C + 16
  slots = E * C2

  x = hidden_states.reshape(T, D)
  ids = input_ids.reshape(T).astype(jnp.int32)

  buf_a, destp_a = _make_routing_dispatch(T, C, C2, 0, 48)(ids, x)
  buf_b, destp_b = _make_routing_dispatch(T, C, C2, E // 2, 48)(ids, x)
  y0 = _make_ffn(C, C2, 1024, 0)(buf_a.reshape(E // 2, C2, D), w1, w2)
  y = _make_ffn_alias(C, C2, 1024, E // 2)(
      y0, buf_b.reshape(E // 2, C2, D), w1, w2)
  out = _make_combine(T, 32)(y.reshape(slots, D), destp_a, destp_b)
  return out.reshape(B, S, D)


# speedup vs baseline: 1.0105x; 1.0105x over previous
import functools

import jax
import jax.numpy as jnp
import numpy as np
from jax import lax
from jax.experimental import pallas as pl
from jax.experimental.pallas import tpu as pltpu
from jax.experimental.pallas import tpu_sc as plsc

E = 64
D = 1024
F = 2048
CAP_FACTOR = 1.25
NC, NS = 2, 16
NW = NC * NS


def _make_routing_dispatch(T, C, C2, slots, chunk):
  per_w = slots // NW
  n_ch = per_w // chunk
  nv = T // 16
  mesh = plsc.VectorSubcoreMesh(core_axis_name="c", subcore_axis_name="s")

  @functools.partial(
      pl.kernel, mesh=mesh,
      compiler_params=pltpu.CompilerParams(needs_layout_passes=False),
      out_type=(jax.ShapeDtypeStruct((slots, D), jnp.float32),
                jax.ShapeDtypeStruct((NW, T), jnp.int32)),
      scratch_types=[
          pltpu.VMEM((T,), jnp.int32),
          pltpu.VMEM((T,), jnp.int32),
          pltpu.VMEM((per_w,), jnp.int32),
          pltpu.VMEM((chunk, D), jnp.float32),
          pltpu.VMEM((chunk, D), jnp.float32),
          pltpu.SemaphoreType.DMA,
          pltpu.SemaphoreType.DMA,
      ],
  )
  def rd(ids_hbm, x_hbm, buf_hbm, destp_hbm, ids_v, destc_v, idx_v,
         rows0, rows1, sem0, sem1):
    cid = lax.axis_index("c")
    sid = lax.axis_index("s")
    w = sid * NC + cid
    base = w * per_w
    e0 = 2 * w
    e1 = 2 * w + 1

    pltpu.sync_copy(ids_hbm, ids_v)

    lanes = lax.iota(jnp.int32, 16)

    def bcast(s):
      return jnp.full((16,), s, jnp.int32)

    def init_body(i, _):
      o = pl.multiple_of(i * 16, 16)
      iv = lanes + bcast(base + i * 16)
      idx_v[pl.ds(o, 16)] = lax.rem(iv, bcast(T))
      return 0

    lax.fori_loop(0, per_w // 16, init_body, 0)

    ec0 = bcast(E)
    e0v = bcast(e0)
    e1v = bcast(e1)
    onev = bcast(1)
    cv = bcast(C)
    d0base = bcast(e0 * C2 + 1)
    d1base = bcast(e1 * C2 + 1)
    c2v = bcast(C2)
    zero = bcast(0)

    dn = lax.GatherDimensionNumbers(offset_dims=(),
                                    collapsed_slice_dims=(0,),
                                    start_index_map=(0,))

    def lgather(s, idx):
      return lax.gather(s, idx[:, None], dn, slice_sizes=(1,),
                        mode=lax.GatherScatterMode.PROMISE_IN_BOUNDS)

    lane15 = bcast(15)

    def body(i, carry):
      c0v, c1v = carry
      o = pl.multiple_of(i * 16, 16)
      v = ids_v[pl.ds(o, 16)]
      ex = lax.rem(v, ec0)
      tok = lanes + bcast(i * 16)
      m0 = ex == e0v
      m1 = ex == e1v
      im0 = jnp.where(m0, onev, zero)
      im1 = jnp.where(m1, onev, zero)

      p0 = plsc.cumsum(im0)
      p1 = plsc.cumsum(im1)
      s0 = c0v + p0 - onev
      s1 = c1v + p1 - onev
      k0 = jnp.logical_and(m0, s0 < cv)
      k1 = jnp.logical_and(m1, s1 < cv)
      plsc.store_scatter(idx_v, [s0], tok, mask=k0)
      plsc.store_scatter(idx_v, [s1 + c2v], tok, mask=k1)
      d0 = d0base + jnp.minimum(s0, cv)
      d1 = d1base + jnp.minimum(s1, cv)
      destc_v[pl.ds(o, 16)] = jnp.where(m0, d0, jnp.where(m1, d1, zero))
      c0v = c0v + lgather(p0, lane15)
      c1v = c1v + lgather(p1, lane15)
      return (c0v, c1v)

    lax.fori_loop(0, nv, body, (zero, zero))

    pltpu.sync_copy(destc_v, destp_hbm.at[w])

    rows = (rows0, rows1)
    sems = (sem0, sem1)
    cps = [None, None]
    cps[0] = pltpu.async_copy(x_hbm.at[idx_v.at[pl.ds(0, chunk)]], rows0, sem0)
    for k in range(n_ch):
      if k + 1 < n_ch:
        cps[(k + 1) % 2] = pltpu.async_copy(
            x_hbm.at[idx_v.at[pl.ds((k + 1) * chunk, chunk)]],
            rows[(k + 1) % 2], sems[(k + 1) % 2])
      cps[k % 2].wait()
      pltpu.sync_copy(rows[k % 2], buf_hbm.at[pl.ds(base + k * chunk, chunk)])

  return rd


def _ffn_body(C, C2, buf_ref, w1_ref, w2_ref, y_ref):
  f = pl.program_id(1)
  b = buf_ref[0]
  h = jnp.maximum(
      lax.dot_general(b, w1_ref[0], (((1,), (0,)), ((), ())),
                      preferred_element_type=jnp.float32), 0.0)
  yp = lax.dot_general(h, w2_ref[0], (((1,), (0,)), ((), ())),
                       preferred_element_type=jnp.float32)
  rowmask = (lax.broadcasted_iota(jnp.int32, (C2, 1), 0) < C).astype(yp.dtype)
  yp = yp * rowmask

  @pl.when(f == 0)
  def _():
    y_ref[0] = yp

  @pl.when(f != 0)
  def _():
    y_ref[0] = y_ref[0] + yp


def _make_ffn(C, C2, fblk):
  nf = F // fblk
  return pl.pallas_call(
      functools.partial(_ffn_body, C, C2),
      grid=(E, nf),
      in_specs=[
          pl.BlockSpec((1, C2, D), lambda e, f: (e, 0, 0)),
          pl.BlockSpec((1, D, fblk), lambda e, f: (e, 0, f)),
          pl.BlockSpec((1, fblk, D), lambda e, f: (e, f, 0)),
      ],
      out_specs=pl.BlockSpec((1, C2, D), lambda e, f: (e, 0, 0)),
      out_shape=jax.ShapeDtypeStruct((E, C2, D), jnp.float32),
      compiler_params=pltpu.CompilerParams(
          dimension_semantics=("arbitrary", "arbitrary")),
  )


def _make_combine(T, chunk):
  per_w = T // NW
  n_ch = per_w // chunk
  mesh = plsc.VectorSubcoreMesh(core_axis_name="c", subcore_axis_name="s")

  @functools.partial(
      pl.kernel, mesh=mesh,
      compiler_params=pltpu.CompilerParams(needs_layout_passes=False),
      out_type=jax.ShapeDtypeStruct((T, D), jnp.float32),
      scratch_types=[
          pltpu.VMEM((NW, per_w), jnp.int32),
          pltpu.VMEM((per_w,), jnp.int32),
          pltpu.VMEM((chunk, D), jnp.float32),
          pltpu.VMEM((chunk, D), jnp.float32),
          pltpu.SemaphoreType.DMA,
          pltpu.SemaphoreType.DMA,
      ],
  )
  def combine(y_hbm, destp_hbm, out_hbm, dp_v, idx_v, rows0, rows1,
              sem0, sem1):
    wid = lax.axis_index("s") * NC + lax.axis_index("c")
    base = wid * per_w
    pltpu.sync_copy(destp_hbm.at[:, pl.ds(base, per_w)], dp_v)

    def sum_body(i, _):
      o = pl.multiple_of(i * 16, 16)
      acc = jnp.full((16,), -1, jnp.int32)
      for s2 in range(NW):
        acc = acc + dp_v[s2, pl.ds(o, 16)]
      idx_v[pl.ds(o, 16)] = acc
      return 0

    lax.fori_loop(0, per_w // 16, sum_body, 0)

    rows = (rows0, rows1)
    sems = (sem0, sem1)
    cps = [None, None]
    cps[0] = pltpu.async_copy(y_hbm.at[idx_v.at[pl.ds(0, chunk)]], rows0, sem0)
    for k in range(n_ch):
      if k + 1 < n_ch:
        cps[(k + 1) % 2] = pltpu.async_copy(
            y_hbm.at[idx_v.at[pl.ds((k + 1) * chunk, chunk)]],
            rows[(k + 1) % 2], sems[(k + 1) % 2])
      cps[k % 2].wait()
      pltpu.sync_copy(rows[k % 2], out_hbm.at[pl.ds(base + k * chunk, chunk)])

  return combine


@jax.jit
def kernel(hidden_states, input_ids, w1, w2):
  B, S, _ = hidden_states.shape
  T = B * S
  C = int(np.ceil(T / E * CAP_FACTOR))
  C2 = C + 16
  slots = E * C2

  x = hidden_states.reshape(T, D)
  ids = input_ids.reshape(T).astype(jnp.int32)

  buf, destp = _make_routing_dispatch(T, C, C2, slots, 48)(ids, x)
  y = _make_ffn(C, C2, 1024)(buf.reshape(E, C2, D), w1, w2)
  out = _make_combine(T, 32)(y.reshape(slots, D), destp)
  return out.reshape(B, S, D)
